# baseline (device time: 173530 ns/iter reference)
import functools

import jax
import jax.numpy as jnp
from jax import lax
from jax.experimental import pallas as pl
from jax.experimental.pallas import tpu as pltpu

N_DEV = 4


def kernel(x, w_mat):
    w_mat = w_mat.astype(jnp.bfloat16)

    m_per, k = x.shape
    _, n_per = w_mat.shape
    m_half = m_per // 2

    def body(x_hbm, w_ref, out_ref,
             xstage, srcR, srcL, load_sems,
             commR_ref, commL_ref,
             sendR_sems, recvR_sems, sendL_sems, recvL_sems,
             amax_ref, amax_send_sems, amax_recv_sems):
        my = lax.axis_index("i")
        left = (my - 1) % N_DEV
        right = (my + 1) % N_DEV

        dma_top = pltpu.make_async_copy(
            x_hbm.at[pl.ds(0, m_half)], xstage, load_sems.at[0])
        dma_top.start()

        barrier_sem = pltpu.get_barrier_semaphore()
        for nbr in (left, right):
            pl.semaphore_signal(
                barrier_sem, inc=1,
                device_id=(nbr,), device_id_type=pl.DeviceIdType.MESH,
            )
        pl.semaphore_wait(barrier_sem, 2)

        def make_hopR(h):
            src = srcR if h == 0 else commR_ref.at[h - 1]
            return pltpu.make_async_remote_copy(
                src_ref=src,
                dst_ref=commR_ref.at[h],
                send_sem=sendR_sems.at[h],
                recv_sem=recvR_sems.at[h],
                device_id=(right,),
                device_id_type=pl.DeviceIdType.MESH,
            )

        def make_hopL(h):
            src = srcL if h == 0 else commL_ref.at[h - 1]
            return pltpu.make_async_remote_copy(
                src_ref=src,
                dst_ref=commL_ref.at[h],
                send_sem=sendL_sems.at[h],
                recv_sem=recvL_sems.at[h],
                device_id=(left,),
                device_id_type=pl.DeviceIdType.MESH,
            )

        dma_top.wait()
        srcR[...] = xstage[...].astype(jnp.bfloat16)
        hopsR = [make_hopR(0)]
        hopsR[0].start()

        dma_bot = pltpu.make_async_copy(
            x_hbm.at[pl.ds(m_half, m_half)], xstage, load_sems.at[0])
        dma_bot.start()
        dma_bot.wait()
        srcL[...] = xstage[...].astype(jnp.bfloat16)
        hopsL = [make_hopL(0)]
        hopsL[0].start()

        amax_parts = []

        def gemm(src, row0):
            yblk = jnp.dot(src, w_ref[...],
                           preferred_element_type=jnp.float32)
            out_ref[pl.ds(row0, yblk.shape[0]), :] = yblk.astype(jnp.bfloat16)
            amax_parts.append(jnp.max(jnp.abs(yblk)))

        gemm(srcR[...], my * m_per)
        gemm(srcL[...], my * m_per + m_half)

        for h in range(N_DEV - 1):
            hopsR[h].wait_recv()
            if h + 1 < N_DEV - 1:
                hopsR.append(make_hopR(h + 1))
                hopsR[h + 1].start()
            hopsL[h].wait_recv()
            if h + 1 < N_DEV - 1:
                hopsL.append(make_hopL(h + 1))
                hopsL[h + 1].start()
            originR = (my - h - 1) % N_DEV
            originL = (my + h + 1) % N_DEV
            gemm(commR_ref[h], originR * m_per)
            gemm(commL_ref[h], originL * m_per + m_half)

        for hop in hopsR + hopsL:
            hop.wait_send()

        local_amax = functools.reduce(jnp.maximum, amax_parts)
        amax_ref[pl.ds(my, 1), :] = jnp.full((1, 128), local_amax,
                                             jnp.float32)

        sends = []
        for d in range(1, N_DEV):
            peer = (my + d) % N_DEV
            s = pltpu.make_async_remote_copy(
                src_ref=amax_ref.at[pl.ds(my, 1)],
                dst_ref=amax_ref.at[pl.ds(my, 1)],
                send_sem=amax_send_sems.at[d - 1],
                recv_sem=amax_recv_sems.at[d - 1],
                device_id=(peer,),
                device_id_type=pl.DeviceIdType.MESH,
            )
            s.start()
            sends.append(s)

        for d in range(1, N_DEV):
            origin = (my - d) % N_DEV
            r = pltpu.make_async_remote_copy(
                src_ref=amax_ref.at[pl.ds(origin, 1)],
                dst_ref=amax_ref.at[pl.ds(origin, 1)],
                send_sem=amax_send_sems.at[d - 1],
                recv_sem=amax_recv_sems.at[d - 1],
                device_id=(origin,),
                device_id_type=pl.DeviceIdType.MESH,
            )
            r.wait_recv()
        for s in sends:
            s.wait_send()

        global_amax = jnp.max(amax_ref[...])
        scale = global_amax / 127.0
        inv_scale = 127.0 / global_amax
        for rb in range(N_DEV * 2):
            y = out_ref[pl.ds(rb * m_half, m_half), :].astype(jnp.float32)
            q = jnp.clip(jnp.round(y * inv_scale), -127.0, 127.0)
            out_ref[pl.ds(rb * m_half, m_half), :] = (
                q * scale).astype(jnp.bfloat16)

    return pl.pallas_call(
        body,
        out_shape=jax.ShapeDtypeStruct((N_DEV * m_per, n_per), jnp.bfloat16),
        in_specs=[
            pl.BlockSpec(memory_space=pl.ANY),
            pl.BlockSpec(memory_space=pltpu.VMEM),
        ],
        out_specs=pl.BlockSpec(memory_space=pltpu.VMEM),
        scratch_shapes=[
            pltpu.VMEM((m_half, k), jnp.float32),
            pltpu.VMEM((m_half, k), jnp.bfloat16),
            pltpu.VMEM((m_half, k), jnp.bfloat16),
            pltpu.SemaphoreType.DMA((1,)),
            pltpu.VMEM((N_DEV - 1, m_half, k), jnp.bfloat16),
            pltpu.VMEM((N_DEV - 1, m_half, k), jnp.bfloat16),
            pltpu.SemaphoreType.DMA((N_DEV - 1,)),
            pltpu.SemaphoreType.DMA((N_DEV - 1,)),
            pltpu.SemaphoreType.DMA((N_DEV - 1,)),
            pltpu.SemaphoreType.DMA((N_DEV - 1,)),
            pltpu.VMEM((N_DEV, 128), jnp.float32),
            pltpu.SemaphoreType.DMA((N_DEV - 1,)),
            pltpu.SemaphoreType.DMA((N_DEV - 1,)),
        ],
        compiler_params=pltpu.CompilerParams(
            collective_id=0,
            vmem_limit_bytes=100 * 1024 * 1024,
        ),
    )(x, w_mat)


# device time: 154171 ns/iter; 1.1256x vs baseline; 1.1256x over previous
import functools

import jax
import jax.numpy as jnp
from jax import lax
from jax.experimental import pallas as pl
from jax.experimental.pallas import tpu as pltpu

N_DEV = 4
SUBS = 4


def kernel(x, w_mat):
    m_per, k = x.shape
    _, n_per = w_mat.shape
    m_half = m_per // 2
    m_sub = m_half // SUBS
    k_half = k // 2

    def body(x_hbm, w_hbm, out_ref,
             xstage, srcR, srcL, wstage, w_bf16, load_sems, wload_sems,
             commR_ref, commL_ref,
             sendR_sems, recvR_sems, sendL_sems, recvL_sems,
             amax_ref, amax_send_sems, amax_recv_sems):
        my = lax.axis_index("i")
        left = (my - 1) % N_DEV
        right = (my + 1) % N_DEV

        xdmas = []
        for s in range(SUBS):
            for half in range(2):
                row = half * m_half + s * m_sub
                d = pltpu.make_async_copy(
                    x_hbm.at[pl.ds(row, m_sub)],
                    xstage.at[pl.ds(row, m_sub)],
                    load_sems.at[2 * s + half])
                d.start()
                xdmas.append(d)
        wdmas = []
        for i in range(2):
            d = pltpu.make_async_copy(
                w_hbm.at[pl.ds(i * k_half, k_half)],
                wstage.at[i],
                wload_sems.at[i])
            d.start()
            wdmas.append(d)

        barrier_sem = pltpu.get_barrier_semaphore()
        for nbr in (left, right):
            pl.semaphore_signal(
                barrier_sem, inc=1,
                device_id=(nbr,), device_id_type=pl.DeviceIdType.MESH,
            )
        pl.semaphore_wait(barrier_sem, 2)

        def make_hop(cw, h, s):
            comm = commR_ref if cw else commL_ref
            src0 = srcR if cw else srcL
            src = (src0.at[pl.ds(s * m_sub, m_sub)] if h == 0
                   else comm.at[h - 1, pl.ds(s * m_sub, m_sub)])
            return pltpu.make_async_remote_copy(
                src_ref=src,
                dst_ref=comm.at[h, pl.ds(s * m_sub, m_sub)],
                send_sem=(sendR_sems if cw else sendL_sems).at[h, s],
                recv_sem=(recvR_sems if cw else recvL_sems).at[h, s],
                device_id=(right if cw else left,),
                device_id_type=pl.DeviceIdType.MESH,
            )

        hops = {}
        for s in range(SUBS):
            for half in range(2):
                row = half * m_half + s * m_sub
                xdmas[2 * s + half].wait()
                dst = srcL if half else srcR
                dst[pl.ds(s * m_sub, m_sub), :] = (
                    xstage[pl.ds(row, m_sub), :].astype(jnp.bfloat16))
                cw = half == 0
                hops[(cw, 0, s)] = make_hop(cw, 0, s)
                hops[(cw, 0, s)].start()

        for i in range(2):
            wdmas[i].wait()
            w_bf16[pl.ds(i * k_half, k_half), :] = (
                wstage[i].astype(jnp.bfloat16))

        amax_parts = []

        def gemm(src, row0):
            yblk = jnp.dot(src, w_bf16[...],
                           preferred_element_type=jnp.float32)
            out_ref[pl.ds(row0, yblk.shape[0]), :] = yblk.astype(jnp.bfloat16)
            amax_parts.append(jnp.max(jnp.abs(yblk)))

        gemm(srcR[...], my * m_per)
        gemm(srcL[...], my * m_per + m_half)

        for h in range(N_DEV - 1):
            originR = (my - h - 1) % N_DEV
            originL = (my + h + 1) % N_DEV
            for s in range(SUBS):
                for cw in (True, False):
                    hops[(cw, h, s)].wait_recv()
                    if h + 1 < N_DEV - 1:
                        hops[(cw, h + 1, s)] = make_hop(cw, h + 1, s)
                        hops[(cw, h + 1, s)].start()
                    comm = commR_ref if cw else commL_ref
                    origin = originR if cw else originL
                    row0 = origin * m_per + (0 if cw else m_half) + s * m_sub
                    gemm(comm[h, pl.ds(s * m_sub, m_sub), :], row0)

        for hop in hops.values():
            hop.wait_send()

        local_amax = functools.reduce(jnp.maximum, amax_parts)
        amax_ref[pl.ds(my, 1), :] = jnp.full((1, 128), local_amax,
                                             jnp.float32)

        sends = []
        for d in range(1, N_DEV):
            peer = (my + d) % N_DEV
            s = pltpu.make_async_remote_copy(
                src_ref=amax_ref.at[pl.ds(my, 1)],
                dst_ref=amax_ref.at[pl.ds(my, 1)],
                send_sem=amax_send_sems.at[d - 1],
                recv_sem=amax_recv_sems.at[d - 1],
                device_id=(peer,),
                device_id_type=pl.DeviceIdType.MESH,
            )
            s.start()
            sends.append(s)

        for d in range(1, N_DEV):
            origin = (my - d) % N_DEV
            r = pltpu.make_async_remote_copy(
                src_ref=amax_ref.at[pl.ds(origin, 1)],
                dst_ref=amax_ref.at[pl.ds(origin, 1)],
                send_sem=amax_send_sems.at[d - 1],
                recv_sem=amax_recv_sems.at[d - 1],
                device_id=(origin,),
                device_id_type=pl.DeviceIdType.MESH,
            )
            r.wait_recv()
        for s in sends:
            s.wait_send()

        global_amax = jnp.max(amax_ref[...])
        scale = global_amax / 127.0
        inv_scale = 127.0 / global_amax
        for rb in range(N_DEV * 2):
            y = out_ref[pl.ds(rb * m_half, m_half), :].astype(jnp.float32)
            q = jnp.clip(jnp.round(y * inv_scale), -127.0, 127.0)
            out_ref[pl.ds(rb * m_half, m_half), :] = (
                q * scale).astype(jnp.bfloat16)

    return pl.pallas_call(
        body,
        out_shape=jax.ShapeDtypeStruct((N_DEV * m_per, n_per), jnp.bfloat16),
        in_specs=[
            pl.BlockSpec(memory_space=pl.ANY),
            pl.BlockSpec(memory_space=pl.ANY),
        ],
        out_specs=pl.BlockSpec(memory_space=pltpu.VMEM),
        scratch_shapes=[
            pltpu.VMEM((m_per, k), jnp.float32),
            pltpu.VMEM((m_half, k), jnp.bfloat16),
            pltpu.VMEM((m_half, k), jnp.bfloat16),
            pltpu.VMEM((2, k_half, n_per), jnp.float32),
            pltpu.VMEM((k, n_per), jnp.bfloat16),
            pltpu.SemaphoreType.DMA((2 * SUBS,)),
            pltpu.SemaphoreType.DMA((2,)),
            pltpu.VMEM((N_DEV - 1, m_half, k), jnp.bfloat16),
            pltpu.VMEM((N_DEV - 1, m_half, k), jnp.bfloat16),
            pltpu.SemaphoreType.DMA((N_DEV - 1, SUBS)),
            pltpu.SemaphoreType.DMA((N_DEV - 1, SUBS)),
            pltpu.SemaphoreType.DMA((N_DEV - 1, SUBS)),
            pltpu.SemaphoreType.DMA((N_DEV - 1, SUBS)),
            pltpu.VMEM((N_DEV, 128), jnp.float32),
            pltpu.SemaphoreType.DMA((N_DEV - 1,)),
            pltpu.SemaphoreType.DMA((N_DEV - 1,)),
        ],
        compiler_params=pltpu.CompilerParams(
            collective_id=0,
            vmem_limit_bytes=100 * 1024 * 1024,
        ),
    )(x, w_mat)
